# Initial kernel scaffold; baseline (speedup 1.0000x reference)
import jax
import jax.numpy as jnp
from jax.experimental import pallas as pl


def kernel(edge):
    raise NotImplementedError("write your pallas kernel here")


# speedup vs baseline: 66.2381x; 66.2381x over previous
import functools

import jax
import jax.numpy as jnp
from jax import lax
from jax.experimental import pallas as pl
from jax.experimental.pallas import tpu as pltpu
from jax.experimental.pallas import tpu_sc as plsc

MAX_ATOMS = 32
MAX_DEGREE = 4
MAX_RINGS = 6
BATCH = 4096

A1 = MAX_ATOMS + 1
EW = MAX_ATOMS * MAX_DEGREE
L = 16
NWORKERS = 32
MOL_PER_W = BATCH // NWORKERS
GROUPS = MOL_PER_W // L


def _setbit(w0, w1, n):
    hi = n >= 32
    sh = jnp.where(hi, 0, n)
    m0 = jnp.where(hi, 0, lax.shift_left(1, sh))
    m1 = jnp.where(hi, 1, 0)
    return w0 | m0, w1 | m1


def _clearbit(w0, w1, n):
    hi = n >= 32
    sh = jnp.where(hi, 0, n)
    m0 = jnp.where(hi, 0, lax.shift_left(1, sh))
    m1 = jnp.where(hi, 1, 0)
    return w0 & ~m0, w1 & ~m1


def _getbit(w0, w1, n):
    hi = n >= 32
    sh = jnp.where(hi, n - 32, n)
    w = jnp.where(hi, w1, w0)
    return lax.shift_right_logical(w, sh) & 1


def _body(edge_hbm, tree_out, ring_out, rings_out,
          edges_v, tree_v, ring_v, rings_v):
    info = plsc.get_sparse_core_info()
    wid = lax.axis_index("s") * info.num_cores + lax.axis_index("c")

    lane = lax.iota(jnp.int32, L)
    ebase = lane * EW
    tbase = lane * A1
    rbase = lane * (2 * A1)
    gbase = lane * (MAX_RINGS * MAX_ATOMS)
    zero = jnp.zeros((L,), jnp.int32)
    mone = jnp.full((L,), -1, jnp.int32)

    def group_body(g, _):
        base = wid * MOL_PER_W + g * L

        pltpu.sync_copy(edge_hbm.at[pl.ds(base * EW, L * EW)], edges_v)

        def z_tree(i, c):
            tree_v[pl.ds(i * L, L)] = zero
            return c

        lax.fori_loop(0, A1, z_tree, 0)

        def z_ring(i, c):
            ring_v[pl.ds(i * L, L)] = zero
            return c

        lax.fori_loop(0, 2 * A1, z_ring, 0)

        def z_rings(i, c):
            rings_v[pl.ds(i * L, L)] = mone
            return c

        lax.fori_loop(0, MAX_RINGS * MAX_ATOMS, z_rings, 0)

        def p1_body(t, carry):
            lnf0, lnf1, nnv0, nnv1, nr = carry
            f0 = ~lnf0 & nnv0
            f1 = ~lnf1 & nnv1 & 1
            low = f0 & jnp.negative(f0)
            fb = lax.bitcast_convert_type(low.astype(jnp.float32), jnp.int32)
            e = (lax.shift_right_logical(fb, 23) & 0xFF) - 127
            v = jnp.where(f0 != 0, e, jnp.where(f1 != 0, 32, 0))
            vpos = v > 0
            for d in range(MAX_DEGREE):
                eoff = jnp.maximum((v - 1) * MAX_DEGREE + d, 0)
                nn_raw = plsc.load_gather(edges_v, [ebase + eoff]) + 1
                nn = jnp.where(vpos, nn_raw, 0)
                s_lnf = _getbit(lnf0, lnf1, nn)
                s_lnv = _getbit(nnv0, nnv1, nn)
                cf = (1 - s_lnf) * s_lnv * jnp.where(nn != 0, 1, 0)
                ok = cf * jnp.where(nr < A1, 1, 0)
                row = jnp.minimum(nr, A1 - 1)
                plsc.addupdate_scatter(ring_v, [rbase + row * 2], v * ok)
                plsc.addupdate_scatter(ring_v, [rbase + row * 2 + 1], nn * ok)
                nr = nr + cf
                plsc.addupdate_scatter(tree_v, [tbase + nn], s_lnf * v)
                lnf0, lnf1 = _clearbit(lnf0, lnf1, nn)
            nnv0, nnv1 = _clearbit(nnv0, nnv1, v)
            return (lnf0, lnf1, nnv0, nnv1, nr)

        init = (jnp.full((L,), -4, jnp.int32),
                jnp.full((L,), 1, jnp.int32),
                jnp.full((L,), -2, jnp.int32),
                jnp.full((L,), 1, jnp.int32),
                zero)
        lax.fori_loop(1, MAX_ATOMS + 1, p1_body, init)

        def ring_body(i, c):
            r0 = plsc.load_gather(ring_v, [rbase + i * 2])
            r1 = plsc.load_gather(ring_v, [rbase + i * 2 + 1])

            b0, b1 = _setbit(zero, zero, r0)

            def w1(j, cc):
                s, a0, a1 = cc
                nxt = plsc.load_gather(tree_v, [tbase + s])
                a0, a1 = _setbit(a0, a1, nxt)
                return (nxt, a0, a1)

            _, vb00, vb01 = lax.fori_loop(0, MAX_ATOMS, w1, (r0, b0, b1))

            vb10, vb11 = _setbit(zero, zero, r1)
            plsc.addupdate_scatter(rings_v, [gbase + i * MAX_ATOMS], r1)

            def w2(j, cc):
                s, a0, a1, aidx = cc
                nxt = plsc.load_gather(tree_v, [tbase + s])
                same = ((vb00 & a0) | (vb01 & a1)) != 0
                ns = jnp.where(same, 0, 1)
                aidx = aidx + ns
                okk = ns * jnp.where(aidx < MAX_ATOMS, 1, 0)
                pos = jnp.minimum(aidx, MAX_ATOMS - 1)
                plsc.addupdate_scatter(
                    rings_v, [gbase + i * MAX_ATOMS + pos], nxt * okk)
                a0, a1 = _setbit(a0, a1, nxt)
                return (nxt, a0, a1, aidx)

            _, vb10, vb11, aidx = lax.fori_loop(
                0, MAX_ATOMS, w2, (r1, vb10, vb11, zero))

            c0, c1 = _setbit(zero, zero, r0)
            aidx = aidx + 1
            okk = jnp.where(aidx < MAX_ATOMS, 1, 0)
            pos = jnp.minimum(aidx, MAX_ATOMS - 1)
            plsc.addupdate_scatter(
                rings_v, [gbase + i * MAX_ATOMS + pos], r0 * okk)

            def w3(j, cc):
                s, a0, a1, aidx = cc
                nxt = plsc.load_gather(tree_v, [tbase + s])
                a0, a1 = _setbit(a0, a1, nxt)
                same = ((a0 & vb10) | (a1 & vb11)) != 0
                ns = jnp.where(same, 0, 1)
                aidx = aidx + ns
                okk = ns * jnp.where(aidx < MAX_ATOMS, 1, 0)
                pos = jnp.minimum(aidx, MAX_ATOMS - 1)
                plsc.addupdate_scatter(
                    rings_v, [gbase + i * MAX_ATOMS + pos], nxt * okk)
                return (nxt, a0, a1, aidx)

            lax.fori_loop(0, MAX_ATOMS, w3, (r0, c0, c1, aidx))
            return c

        lax.fori_loop(0, MAX_RINGS, ring_body, 0)

        pltpu.sync_copy(tree_v, tree_out.at[pl.ds(base * A1, L * A1)])
        pltpu.sync_copy(ring_v, ring_out.at[pl.ds(base * 2 * A1, L * 2 * A1)])
        pltpu.sync_copy(
            rings_v,
            rings_out.at[pl.ds(base * MAX_RINGS * MAX_ATOMS,
                               L * MAX_RINGS * MAX_ATOMS)])
        return _

    lax.fori_loop(0, GROUPS, group_body, 0)


@jax.jit
def kernel(edge):
    B = edge.shape[0]
    flat = edge.astype(jnp.int32).reshape(B * EW)
    call = pl.kernel(
        _body,
        out_type=(
            jax.ShapeDtypeStruct((B * A1,), jnp.int32),
            jax.ShapeDtypeStruct((B * 2 * A1,), jnp.int32),
            jax.ShapeDtypeStruct((B * MAX_RINGS * MAX_ATOMS,), jnp.int32),
        ),
        mesh=plsc.VectorSubcoreMesh(core_axis_name="c", subcore_axis_name="s"),
        scratch_types=(
            pltpu.VMEM((L * EW,), jnp.int32),
            pltpu.VMEM((L * A1,), jnp.int32),
            pltpu.VMEM((L * 2 * A1,), jnp.int32),
            pltpu.VMEM((L * MAX_RINGS * MAX_ATOMS,), jnp.int32),
        ),
        compiler_params=pltpu.CompilerParams(needs_layout_passes=False),
    )
    tree_f, ring_f, rings_f = call(flat)
    return (tree_f.reshape(B, A1),
            ring_f.reshape(B, A1, 2),
            rings_f.reshape(B, MAX_RINGS, MAX_ATOMS))


# speedup vs baseline: 77.2573x; 1.1664x over previous
import functools

import jax
import jax.numpy as jnp
from jax import lax
from jax.experimental import pallas as pl
from jax.experimental.pallas import tpu as pltpu
from jax.experimental.pallas import tpu_sc as plsc

MAX_ATOMS = 32
MAX_DEGREE = 4
MAX_RINGS = 6
BATCH = 4096

A1 = MAX_ATOMS + 1
EW = MAX_ATOMS * MAX_DEGREE
L = 16
NWORKERS = 32
MOL_PER_W = BATCH // NWORKERS
GROUPS = MOL_PER_W // L


def _setbit(w0, w1, n):
    hi = n >= 32
    sh = jnp.where(hi, 0, n)
    m0 = jnp.where(hi, 0, lax.shift_left(1, sh))
    m1 = jnp.where(hi, 1, 0)
    return w0 | m0, w1 | m1


def _clearbit(w0, w1, n):
    hi = n >= 32
    sh = jnp.where(hi, 0, n)
    m0 = jnp.where(hi, 0, lax.shift_left(1, sh))
    m1 = jnp.where(hi, 1, 0)
    return w0 & ~m0, w1 & ~m1


def _getbit(w0, w1, n):
    hi = n >= 32
    sh = jnp.where(hi, n - 32, n)
    w = jnp.where(hi, w1, w0)
    return lax.shift_right_logical(w, sh) & 1


def _body(edge_hbm, tree_out, ring_out, rings_out,
          edges_v, tree_v, ring_v, rings_v):
    info = plsc.get_sparse_core_info()
    wid = lax.axis_index("s") * info.num_cores + lax.axis_index("c")

    lane = lax.iota(jnp.int32, L)
    ebase = lane * EW
    tbase = lane * A1
    rbase = lane * (2 * A1)
    gbase = lane * (MAX_RINGS * MAX_ATOMS)
    zero = jnp.zeros((L,), jnp.int32)
    mone = jnp.full((L,), -1, jnp.int32)

    def group_body(g, gcarry):
        base = wid * MOL_PER_W + g * L

        pltpu.sync_copy(edge_hbm.at[pl.ds(base * EW, L * EW)], edges_v)

        for i in range(A1):
            tree_v[pl.ds(i * L, L)] = zero
        for i in range(2 * A1):
            ring_v[pl.ds(i * L, L)] = zero
        for i in range(MAX_RINGS * MAX_ATOMS):
            rings_v[pl.ds(i * L, L)] = mone

        def p1_body(t, carry):
            lnf0, lnf1, nnv0, nnv1, nr = carry
            f0 = ~lnf0 & nnv0
            f1 = ~lnf1 & nnv1 & 1
            low = f0 & jnp.negative(f0)
            fb = lax.bitcast_convert_type(low.astype(jnp.float32), jnp.int32)
            e = (lax.shift_right_logical(fb, 23) & 0xFF) - 127
            v = jnp.where(f0 != 0, e, jnp.where(f1 != 0, 32, 0))
            vpos = v > 0
            for d in range(MAX_DEGREE):
                eoff = jnp.maximum((v - 1) * MAX_DEGREE + d, 0)
                nn_raw = plsc.load_gather(edges_v, [ebase + eoff]) + 1
                nn = jnp.where(vpos, nn_raw, 0)
                s_lnf = _getbit(lnf0, lnf1, nn)
                s_lnv = _getbit(nnv0, nnv1, nn)
                cf = (1 - s_lnf) * s_lnv * jnp.where(nn != 0, 1, 0)
                ok = cf * jnp.where(nr < A1, 1, 0)
                row = jnp.minimum(nr, A1 - 1)
                plsc.addupdate_scatter(ring_v, [rbase + row * 2], v * ok)
                plsc.addupdate_scatter(ring_v, [rbase + row * 2 + 1], nn * ok)
                nr = nr + cf
                plsc.addupdate_scatter(tree_v, [tbase + nn], s_lnf * v)
                lnf0, lnf1 = _clearbit(lnf0, lnf1, nn)
            nnv0, nnv1 = _clearbit(nnv0, nnv1, v)
            return (lnf0, lnf1, nnv0, nnv1, nr)

        init = (jnp.full((L,), -4, jnp.int32),
                jnp.full((L,), 1, jnp.int32),
                jnp.full((L,), -2, jnp.int32),
                jnp.full((L,), 1, jnp.int32),
                zero)
        lax.fori_loop(1, MAX_ATOMS + 1, p1_body, init)

        NR = MAX_RINGS

        def alive(j, ss):
            acc = ss[0]
            for i in range(1, NR):
                acc = acc | ss[i]
            return (j < MAX_ATOMS) & jnp.any(acc != 0)

        R0 = [plsc.load_gather(ring_v, [rbase + i * 2]) for i in range(NR)]
        R1 = [plsc.load_gather(ring_v, [rbase + i * 2 + 1])
              for i in range(NR)]

        b0s, b1s = zip(*[_setbit(zero, zero, R0[i]) for i in range(NR)])

        def w1_cond(c):
            return alive(c[0], c[1])

        def w1_body(c):
            j, ss, a0s, a1s = c
            ns_, n0_, n1_ = [], [], []
            for i in range(NR):
                nxt = plsc.load_gather(tree_v, [tbase + ss[i]])
                a0, a1 = _setbit(a0s[i], a1s[i], nxt)
                ns_.append(nxt)
                n0_.append(a0)
                n1_.append(a1)
            return (j + 1, tuple(ns_), tuple(n0_), tuple(n1_))

        _, _, VB00, VB01 = lax.while_loop(
            w1_cond, w1_body, (jnp.int32(0), tuple(R0), b0s, b1s))

        c0s, c1s = zip(*[_setbit(zero, zero, R1[i]) for i in range(NR)])
        for i in range(NR):
            plsc.addupdate_scatter(rings_v, [gbase + i * MAX_ATOMS], R1[i])

        def w2_cond(c):
            return alive(c[0], c[1])

        def w2_body(c):
            j, ss, a0s, a1s, aidxs = c
            ns_, n0_, n1_, na_ = [], [], [], []
            for i in range(NR):
                nxt = plsc.load_gather(tree_v, [tbase + ss[i]])
                same = ((VB00[i] & a0s[i]) | (VB01[i] & a1s[i])) != 0
                ns = jnp.where(same, 0, 1)
                aidx = aidxs[i] + ns
                okk = ns * jnp.where(aidx < MAX_ATOMS, 1, 0)
                pos = jnp.minimum(aidx, MAX_ATOMS - 1)
                plsc.addupdate_scatter(
                    rings_v, [gbase + i * MAX_ATOMS + pos], nxt * okk)
                a0, a1 = _setbit(a0s[i], a1s[i], nxt)
                ns_.append(nxt)
                n0_.append(a0)
                n1_.append(a1)
                na_.append(aidx)
            return (j + 1, tuple(ns_), tuple(n0_), tuple(n1_), tuple(na_))

        _, _, VB10, VB11, AIDX = lax.while_loop(
            w2_cond, w2_body,
            (jnp.int32(0), tuple(R1), c0s, c1s, (zero,) * NR))

        d0s, d1s = zip(*[_setbit(zero, zero, R0[i]) for i in range(NR)])
        aidx2 = []
        for i in range(NR):
            aidx = AIDX[i] + 1
            okk = jnp.where(aidx < MAX_ATOMS, 1, 0)
            pos = jnp.minimum(aidx, MAX_ATOMS - 1)
            plsc.addupdate_scatter(
                rings_v, [gbase + i * MAX_ATOMS + pos], R0[i] * okk)
            aidx2.append(aidx)

        def w3_cond(c):
            return alive(c[0], c[1])

        def w3_body(c):
            j, ss, a0s, a1s, aidxs = c
            ns_, n0_, n1_, na_ = [], [], [], []
            for i in range(NR):
                nxt = plsc.load_gather(tree_v, [tbase + ss[i]])
                a0, a1 = _setbit(a0s[i], a1s[i], nxt)
                same = ((a0 & VB10[i]) | (a1 & VB11[i])) != 0
                ns = jnp.where(same, 0, 1)
                aidx = aidxs[i] + ns
                okk = ns * jnp.where(aidx < MAX_ATOMS, 1, 0)
                pos = jnp.minimum(aidx, MAX_ATOMS - 1)
                plsc.addupdate_scatter(
                    rings_v, [gbase + i * MAX_ATOMS + pos], nxt * okk)
                ns_.append(nxt)
                n0_.append(a0)
                n1_.append(a1)
                na_.append(aidx)
            return (j + 1, tuple(ns_), tuple(n0_), tuple(n1_), tuple(na_))

        lax.while_loop(
            w3_cond, w3_body,
            (jnp.int32(0), tuple(R0), d0s, d1s, tuple(aidx2)))

        pltpu.sync_copy(tree_v, tree_out.at[pl.ds(base * A1, L * A1)])
        pltpu.sync_copy(ring_v, ring_out.at[pl.ds(base * 2 * A1, L * 2 * A1)])
        pltpu.sync_copy(
            rings_v,
            rings_out.at[pl.ds(base * MAX_RINGS * MAX_ATOMS,
                               L * MAX_RINGS * MAX_ATOMS)])
        return gcarry

    lax.fori_loop(0, GROUPS, group_body, 0)


@jax.jit
def kernel(edge):
    B = edge.shape[0]
    flat = edge.astype(jnp.int32).reshape(B * EW)
    call = pl.kernel(
        _body,
        out_type=(
            jax.ShapeDtypeStruct((B * A1,), jnp.int32),
            jax.ShapeDtypeStruct((B * 2 * A1,), jnp.int32),
            jax.ShapeDtypeStruct((B * MAX_RINGS * MAX_ATOMS,), jnp.int32),
        ),
        mesh=plsc.VectorSubcoreMesh(core_axis_name="c", subcore_axis_name="s"),
        scratch_types=(
            pltpu.VMEM((L * EW,), jnp.int32),
            pltpu.VMEM((L * A1,), jnp.int32),
            pltpu.VMEM((L * 2 * A1,), jnp.int32),
            pltpu.VMEM((L * MAX_RINGS * MAX_ATOMS,), jnp.int32),
        ),
        compiler_params=pltpu.CompilerParams(needs_layout_passes=False),
    )
    tree_f, ring_f, rings_f = call(flat)
    return (tree_f.reshape(B, A1),
            ring_f.reshape(B, A1, 2),
            rings_f.reshape(B, MAX_RINGS, MAX_ATOMS))
